# baseline (speedup 1.0000x reference)
import functools

import jax
import jax.numpy as jnp
from jax import lax
from jax.experimental import pallas as pl
from jax.experimental.pallas import tpu as pltpu
from jax.experimental.pallas import tpu_sc as plsc

MAXLEN = 200
EMBED = 32
BATCH = 4096

SEQS_PER_CHUNK = 4
CHUNK_ROWS = SEQS_PER_CHUNK * MAXLEN
GSPLIT = 4
GROWS = CHUNK_ROWS // GSPLIT


def kernel(x, token_table, pos_table):
    info = plsc.get_sparse_core_info()
    nc, ns = info.num_cores, info.num_subcores
    nw = nc * ns
    seqs_per_w = BATCH // nw
    n_chunks = seqs_per_w // SEQS_PER_CHUNK

    x_3d = x.reshape(-1).astype(jnp.int32).reshape(
        nw * n_chunks, GSPLIT, GROWS)

    mesh = plsc.VectorSubcoreMesh(core_axis_name="c", subcore_axis_name="s")

    @functools.partial(
        pl.kernel,
        mesh=mesh,
        out_type=jax.ShapeDtypeStruct((BATCH * MAXLEN, EMBED), jnp.float32),
        scratch_types=[
            pltpu.VMEM((n_chunks, GSPLIT, GROWS), jnp.int32),
            pltpu.VMEM((CHUNK_ROWS, EMBED), jnp.float32),
            pltpu.VMEM((CHUNK_ROWS, EMBED), jnp.float32),
            pltpu.VMEM((CHUNK_ROWS, EMBED), jnp.float32),
            pltpu.SemaphoreType.DMA,
            pltpu.SemaphoreType.DMA,
            pltpu.SemaphoreType.DMA,
            pltpu.SemaphoreType.DMA,
        ],
        compiler_params=pltpu.CompilerParams(use_tc_tiling_on_sc=False),
    )
    def emb_kernel(x_hbm, tok_hbm, pos_hbm, out_hbm,
                   idxall_v, rows0_v, rows1_v, posrep_v,
                   gsem0, gsem1, osem0, osem1):
        wid = lax.axis_index("s") * nc + lax.axis_index("c")
        base_row = wid * seqs_per_w * MAXLEN
        rows = (rows0_v, rows1_v)
        gsem = (gsem0, gsem1)
        osem = (osem0, osem1)

        def out_slice(c):
            return out_hbm.at[pl.ds(base_row + c * CHUNK_ROWS, CHUNK_ROWS)]

        def start_gather(c, buf):
            for g in range(GSPLIT):
                pltpu.async_copy(
                    tok_hbm.at[idxall_v.at[c, g]],
                    rows[buf].at[pl.ds(g * GROWS, GROWS)], gsem[buf])

        def wait_gather(c, buf):
            for g in range(GSPLIT):
                pltpu.make_async_copy(
                    tok_hbm.at[idxall_v.at[c, g]],
                    rows[buf].at[pl.ds(g * GROWS, GROWS)], gsem[buf]).wait()

        def add_pos(buf):
            rbuf = rows[buf]

            def add_body(r, carry):
                for h in range(EMBED // 16):
                    sl = pl.ds(h * 16, 16)
                    rbuf[r, sl] = rbuf[r, sl] + posrep_v[r, sl]
                return carry

            lax.fori_loop(0, CHUNK_ROWS, add_body, 0, unroll=8)

        pltpu.sync_copy(x_hbm.at[pl.ds(wid * n_chunks, n_chunks)], idxall_v)
        for i in range(SEQS_PER_CHUNK):
            pltpu.sync_copy(pos_hbm, posrep_v.at[pl.ds(i * MAXLEN, MAXLEN)])

        start_gather(0, 0)
        start_gather(1, 1)
        wait_gather(0, 0)
        add_pos(0)
        pltpu.async_copy(rows0_v, out_slice(0), osem0)

        def pair_body(c2, carry):
            for i, buf in ((0, 1), (1, 0)):
                c = 2 * c2 + 1 + i
                other = 1 - buf
                pltpu.make_async_copy(
                    rows[other], out_slice(c - 1), osem[other]).wait()
                start_gather(c + 1, other)
                wait_gather(c, buf)
                add_pos(buf)
                pltpu.async_copy(rows[buf], out_slice(c), osem[buf])
            return carry

        lax.fori_loop(0, (n_chunks - 2) // 2, pair_body, 0)

        last = n_chunks - 1
        pltpu.make_async_copy(rows0_v, out_slice(last - 1), osem0).wait()
        wait_gather(last, 1)
        add_pos(1)
        pltpu.sync_copy(rows1_v, out_slice(last))

    out = emb_kernel(x_3d, token_table, pos_table)
    return out.reshape(BATCH, MAXLEN, EMBED)


# speedup vs baseline: 1.3094x; 1.3094x over previous
import functools

import jax
import jax.numpy as jnp
from jax import lax
from jax.experimental import pallas as pl
from jax.experimental.pallas import tpu as pltpu
from jax.experimental.pallas import tpu_sc as plsc

MAXLEN = 200
EMBED = 32
BATCH = 4096

SEQS_PER_CHUNK = 2
CHUNK_ROWS = SEQS_PER_CHUNK * MAXLEN
NBUF = 4


def kernel(x, token_table, pos_table):
    info = plsc.get_sparse_core_info()
    nc, ns = info.num_cores, info.num_subcores
    nw = nc * ns
    seqs_per_w = BATCH // nw
    n_chunks = seqs_per_w // SEQS_PER_CHUNK

    x_2d = x.reshape(-1).astype(jnp.int32).reshape(nw * n_chunks, CHUNK_ROWS)

    mesh = plsc.VectorSubcoreMesh(core_axis_name="c", subcore_axis_name="s")

    @functools.partial(
        pl.kernel,
        mesh=mesh,
        out_type=jax.ShapeDtypeStruct((BATCH * MAXLEN, EMBED), jnp.float32),
        scratch_types=[
            pltpu.VMEM((n_chunks, CHUNK_ROWS), jnp.int32),
            pltpu.VMEM((NBUF, CHUNK_ROWS, EMBED), jnp.float32),
            pltpu.VMEM((CHUNK_ROWS, EMBED), jnp.float32),
            [pltpu.SemaphoreType.DMA] * NBUF,
            [pltpu.SemaphoreType.DMA] * NBUF,
        ],
        compiler_params=pltpu.CompilerParams(use_tc_tiling_on_sc=False),
    )
    def emb_kernel(x_hbm, tok_hbm, pos_hbm, out_hbm,
                   idxall_v, rowsbuf_v, posrep_v, gsems, osems):
        wid = lax.axis_index("s") * nc + lax.axis_index("c")
        base_row = wid * seqs_per_w * MAXLEN

        def out_slice(c):
            return out_hbm.at[pl.ds(base_row + c * CHUNK_ROWS, CHUNK_ROWS)]

        def start_gather(c, buf):
            pltpu.async_copy(
                tok_hbm.at[idxall_v.at[c]], rowsbuf_v.at[buf], gsems[buf])

        def wait_gather(c, buf):
            pltpu.make_async_copy(
                tok_hbm.at[idxall_v.at[c]], rowsbuf_v.at[buf],
                gsems[buf]).wait()

        def wait_write(c, buf):
            pltpu.make_async_copy(
                rowsbuf_v.at[buf], out_slice(c), osems[buf]).wait()

        def add_pos(buf):
            @plsc.parallel_loop(0, CHUNK_ROWS, unroll=8)
            def add_body(r):
                for h in range(EMBED // 16):
                    sl = pl.ds(h * 16, 16)
                    rowsbuf_v[buf, r, sl] = (
                        rowsbuf_v[buf, r, sl] + posrep_v[r, sl])

        pltpu.sync_copy(x_hbm.at[pl.ds(wid * n_chunks, n_chunks)], idxall_v)
        for i in range(SEQS_PER_CHUNK):
            pltpu.sync_copy(pos_hbm, posrep_v.at[pl.ds(i * MAXLEN, MAXLEN)])

        for c in range(NBUF - 1):
            start_gather(c, c)

        start_gather(3, 3)
        wait_gather(0, 0)
        add_pos(0)
        pltpu.async_copy(rowsbuf_v.at[0], out_slice(0), osems[0])

        def quad_body(q, carry):
            for b in range(NBUF):
                c = 4 * q + 1 + b
                buf = (1 + b) % NBUF
                prev = b % NBUF
                wait_write(c - 1, prev)
                start_gather(c + 3, prev)
                wait_gather(c, buf)
                add_pos(buf)
                pltpu.async_copy(rowsbuf_v.at[buf], out_slice(c), osems[buf])
            return carry

        lax.fori_loop(0, (n_chunks - 4) // 4, quad_body, 0)

        for c in range(n_chunks - 3, n_chunks):
            buf = c % NBUF
            prev = (c - 1) % NBUF
            wait_write(c - 1, prev)
            wait_gather(c, buf)
            add_pos(buf)
            pltpu.async_copy(rowsbuf_v.at[buf], out_slice(c), osems[buf])
        wait_write(n_chunks - 1, (n_chunks - 1) % NBUF)

    out = emb_kernel(x_2d, token_table, pos_table)
    return out.reshape(BATCH, MAXLEN, EMBED)
